# Initial kernel scaffold; baseline (speedup 1.0000x reference)
import jax
import jax.numpy as jnp
from jax.experimental import pallas as pl


def kernel(feature_indices_0, feature_values_0, feature_indices_1, feature_values_1, weight, bias):
    raise NotImplementedError("write your pallas kernel here")


# speedup vs baseline: 7.9231x; 7.9231x over previous
import functools

import jax
import jax.numpy as jnp
from jax import lax
from jax.experimental import pallas as pl
from jax.experimental.pallas import tpu as pltpu
from jax.experimental.pallas import tpu_sc as plsc

B, A, V, O = 4096, 50, 100000, 128
NC, NS = 2, 16
NW = NC * NS
SPT = B // NW
NCHUNK = O // 16


def _accumulate(s, val_v, rows_v, bias_v, out_v):
    vals = []
    for g in (0, 16, 32, 34):
        vgrp = val_v[s, pl.ds(g, 16)]
        lo = len(vals)
        for k in range(16):
            if g + k >= lo and g + k < A:
                vals.append(vgrp[k])
    accs = [bias_v[pl.ds(c * 16, 16)] for c in range(NCHUNK)]
    for j in range(A):
        for c in range(NCHUNK):
            accs[c] = accs[c] + vals[j] * rows_v[j, pl.ds(c * 16, 16)]
    for c in range(NCHUNK):
        out_v[s, pl.ds(c * 16, 16)] = accs[c]


def kernel(feature_indices_0, feature_values_0, feature_indices_1,
           feature_values_1, weight, bias):
    mesh = plsc.VectorSubcoreMesh(core_axis_name="c", subcore_axis_name="s")
    out_sds = jax.ShapeDtypeStruct((B, O), jnp.float32)

    @functools.partial(
        pl.kernel,
        out_type=(out_sds, out_sds),
        mesh=mesh,
        scratch_types=[
            pltpu.VMEM((SPT, A), jnp.int32),
            pltpu.VMEM((SPT, A), jnp.float32),
            pltpu.VMEM((A, O), jnp.float32),
            pltpu.VMEM((A, O), jnp.float32),
            pltpu.VMEM((SPT, O), jnp.float32),
            pltpu.VMEM((O,), jnp.float32),
            pltpu.SemaphoreType.DMA,
            pltpu.SemaphoreType.DMA,
            pltpu.SemaphoreType.DMA,
        ],
    )
    def run(fi0_h, fv0_h, fi1_h, fv1_h, w_h, b_h, out0_h, out1_h,
            idx_v, val_v, rows0_v, rows1_v, out_v, bias_v,
            sem, semr0, semr1):
        wid = lax.axis_index("s") * NC + lax.axis_index("c")
        base = wid * SPT
        pltpu.sync_copy(b_h, bias_v)
        for fi_h, fv_h, out_h in ((fi0_h, fv0_h, out0_h),
                                  (fi1_h, fv1_h, out1_h)):
            pltpu.async_copy(fi_h.at[pl.ds(base, SPT)], idx_v, sem).wait()
            pltpu.async_copy(fv_h.at[pl.ds(base, SPT)], val_v, sem).wait()
            pltpu.async_copy(w_h.at[idx_v.at[0]], rows0_v, semr0)

            @pl.loop(0, SPT, step=2)
            def _(s):
                pltpu.async_copy(w_h.at[idx_v.at[s + 1]], rows1_v, semr1)
                pltpu.make_async_copy(w_h.at[idx_v.at[s]], rows0_v,
                                      semr0).wait()
                _accumulate(s, val_v, rows0_v, bias_v, out_v)

                @pl.when(s + 2 < SPT)
                def _():
                    pltpu.async_copy(w_h.at[idx_v.at[s + 2]], rows0_v, semr0)

                pltpu.make_async_copy(w_h.at[idx_v.at[s + 1]], rows1_v,
                                      semr1).wait()
                _accumulate(s + 1, val_v, rows1_v, bias_v, out_v)

            pltpu.sync_copy(out_v, out_h.at[pl.ds(base, SPT)])

    return run(feature_indices_0, feature_values_0, feature_indices_1,
               feature_values_1, weight, bias)
